# baseline (speedup 1.0000x reference)
import functools

import jax
import jax.numpy as jnp
from jax import lax
from jax.experimental import pallas as pl
from jax.experimental.pallas import tpu as pltpu
from jax.experimental.pallas import tpu_sc as plsc

N = 50000
F = 32
H = 16
NPAD = 51200
E = 800000
GROUP = 544
G = 96
EPW = GROUP * G
EPAD = EPW * 16
RPT = NPAD // 16


def _agg_body(y_hbm, src_hbm, dst_hbm, zero_hbm, agg_hbm,
              agg_s, srcres, dst_a, dst_b, rows_a, rows_b, gsem, ssem, isem):
    c = lax.axis_index("c")
    s = lax.axis_index("s")

    pltpu.sync_copy(zero_hbm.at[pl.ds(s * RPT, RPT)],
                    agg_s.at[pl.ds(s * RPT, RPT)])
    pltpu.sync_copy(src_hbm.at[s], srcres)
    plsc.subcore_barrier()

    yh = y_hbm.at[c]
    dh = dst_hbm.at[s]

    def pair(kk, carry):
        g = 2 * kk
        da = pltpu.async_copy(dh.at[g], dst_a, isem)
        db = pltpu.async_copy(dh.at[g + 1], dst_b, isem)
        ga = pltpu.async_copy(yh.at[srcres.at[g]], rows_a, gsem)
        gb = pltpu.async_copy(yh.at[srcres.at[g + 1]], rows_b, gsem)
        ga.wait()
        da.wait()
        sa = pltpu.async_copy(rows_a, agg_s.at[dst_a], ssem, add=True)
        gb.wait()
        db.wait()
        sb = pltpu.async_copy(rows_b, agg_s.at[dst_b], ssem, add=True)
        sa.wait()
        sb.wait()
        return carry

    lax.fori_loop(0, G // 2, pair, 0)
    plsc.subcore_barrier()
    pltpu.sync_copy(agg_s.at[pl.ds(s * RPT, RPT)],
                    agg_hbm.at[c, pl.ds(s * RPT, RPT)])


_agg_call = pl.kernel(
    _agg_body,
    out_type=jax.ShapeDtypeStruct((2, NPAD, H), jnp.float32),
    mesh=plsc.VectorSubcoreMesh(core_axis_name="c", subcore_axis_name="s"),
    scratch_types=[
        pltpu.VMEM_SHARED((NPAD, H), jnp.float32),
        pltpu.VMEM((G, GROUP), jnp.int32),
        pltpu.VMEM((GROUP,), jnp.int32),
        pltpu.VMEM((GROUP,), jnp.int32),
        pltpu.VMEM((GROUP, H), jnp.float32),
        pltpu.VMEM((GROUP, H), jnp.float32),
        pltpu.SemaphoreType.DMA,
        pltpu.SemaphoreType.DMA,
        pltpu.SemaphoreType.DMA,
    ],
    compiler_params=pltpu.CompilerParams(use_tc_tiling_on_sc=False),
)


def _inorm(x, g, bt):
    m = jnp.mean(x)
    v = jnp.mean((x - m) ** 2)
    return (x - m) / jnp.sqrt(v + 0.001) * g + bt


def kernel(graph_features, encoder_projection, prev_results, edge_index,
           proc_Wr, proc_Wroot, proc_b, d0_Wr, d0_Wroot, d0_b,
           dW_rel, dW_root, d_b, gammas, betas, out_Wr, out_Wroot, out_b):
    src = edge_index[0]
    dst = edge_index[1]
    pad = EPAD - E
    ar = jnp.arange(pad, dtype=jnp.int32)
    srcp = jnp.concatenate([src, (ar * 131) % N]).reshape(16, G, GROUP)
    dstp = jnp.concatenate([dst, N + ar % (NPAD - N)]).reshape(16, G, GROUP)
    zero = jnp.zeros((NPAD, H), jnp.float32)

    def aggregate(y):
        yp = jnp.stack([y[:, :H], y[:, H:]])
        agg2 = _agg_call(yp, srcp, dstp, zero)
        return jnp.concatenate([agg2[0, :N], agg2[1, :N]], axis=1)

    z = aggregate(graph_features) @ proc_Wr.T + proc_b \
        + graph_features @ proc_Wroot.T
    x0 = jax.nn.relu(z)
    cat = jnp.concatenate([x0, encoder_projection, prev_results], axis=-1)

    def conv32(x, Wr, Wroot, b):
        return aggregate(x) @ Wr.T + b + x @ Wroot.T

    pre = cat @ d0_Wr.T
    z = aggregate(pre) + cat @ d0_Wroot.T + d0_b
    x = _inorm(z, gammas[0], betas[0])
    x1 = x
    x = _inorm(conv32(x, dW_rel[0], dW_root[0], d_b[0]), gammas[1], betas[1])
    x = jax.nn.relu(x)
    x = _inorm(conv32(x, dW_rel[1], dW_root[1], d_b[1]), gammas[2], betas[2])
    x = jax.nn.relu((x + x1) / 2.0)

    for bi in (1, 2):
        i0 = 3 * bi - 1
        x = _inorm(conv32(x, dW_rel[i0], dW_root[i0], d_b[i0]),
                   gammas[3 * bi], betas[3 * bi])
        x1 = x
        x = _inorm(conv32(x, dW_rel[i0 + 1], dW_root[i0 + 1], d_b[i0 + 1]),
                   gammas[3 * bi + 1], betas[3 * bi + 1])
        x = jax.nn.relu(x)
        x = _inorm(conv32(x, dW_rel[i0 + 2], dW_root[i0 + 2], d_b[i0 + 2]),
                   gammas[3 * bi + 2], betas[3 * bi + 2])
        x = jax.nn.relu((x + x1) / 2.0)

    res = aggregate(x) @ out_Wr.T + out_b + x @ out_Wroot.T + prev_results
    return (x, res)


# speedup vs baseline: 1.0189x; 1.0189x over previous
import jax
import jax.numpy as jnp
from jax import lax
from jax.experimental import pallas as pl
from jax.experimental.pallas import tpu as pltpu
from jax.experimental.pallas import tpu_sc as plsc

N = 50000
F = 32
NPAD = 51200
E = 800000
GROUP = 816
G = 32
EPW = GROUP * G
EPAD = EPW * 32
RPT = NPAD // 16


def _agg_body(y_hbm, idx_hbm, zero_hbm, agg_hbm,
              agg_s, idxb, rows, gsem, ssem):
    c = lax.axis_index("c")
    s = lax.axis_index("s")
    wid = c * 16 + s

    pltpu.sync_copy(zero_hbm.at[pl.ds(s * RPT, RPT)],
                    agg_s.at[pl.ds(s * RPT, RPT)])
    plsc.subcore_barrier()

    gbase = wid * G

    def grp(g, carry):
        pltpu.sync_copy(idx_hbm.at[gbase + g], idxb)
        pltpu.async_copy(y_hbm.at[idxb.at[0]], rows, gsem).wait()
        pltpu.async_copy(rows, agg_s.at[idxb.at[1]], ssem, add=True).wait()
        return carry

    lax.fori_loop(0, G, grp, 0)
    plsc.subcore_barrier()
    pltpu.sync_copy(agg_s.at[pl.ds(s * RPT, RPT)],
                    agg_hbm.at[c, pl.ds(s * RPT, RPT)])


_agg_call = pl.kernel(
    _agg_body,
    out_type=jax.ShapeDtypeStruct((2, NPAD, F), jnp.float32),
    mesh=plsc.VectorSubcoreMesh(core_axis_name="c", subcore_axis_name="s"),
    scratch_types=[
        pltpu.VMEM_SHARED((NPAD, F), jnp.float32),
        pltpu.VMEM((2, GROUP), jnp.int32),
        pltpu.VMEM((GROUP, F), jnp.float32),
        pltpu.SemaphoreType.DMA,
        pltpu.SemaphoreType.DMA,
    ],
    compiler_params=pltpu.CompilerParams(use_tc_tiling_on_sc=False),
)


def _inorm(x, g, bt):
    m = jnp.mean(x)
    v = jnp.mean((x - m) ** 2)
    return (x - m) / jnp.sqrt(v + 0.001) * g + bt


def kernel(graph_features, encoder_projection, prev_results, edge_index,
           proc_Wr, proc_Wroot, proc_b, d0_Wr, d0_Wroot, d0_b,
           dW_rel, dW_root, d_b, gammas, betas, out_Wr, out_Wroot, out_b):
    src = edge_index[0]
    dst = edge_index[1]
    pad = EPAD - E
    ar = jnp.arange(pad, dtype=jnp.int32)
    srcp = jnp.concatenate([src, (ar * 131) % N]).reshape(-1, 1, GROUP)
    dstp = jnp.concatenate([dst, N + ar % (NPAD - N)]).reshape(-1, 1, GROUP)
    idx_all = jnp.concatenate([srcp, dstp], axis=1)
    zero = jnp.zeros((NPAD, F), jnp.float32)

    def aggregate(y):
        agg2 = _agg_call(y, idx_all, zero)
        return agg2[0, :N] + agg2[1, :N]

    z = aggregate(graph_features) @ proc_Wr.T + proc_b \
        + graph_features @ proc_Wroot.T
    x0 = jax.nn.relu(z)
    cat = jnp.concatenate([x0, encoder_projection, prev_results], axis=-1)

    def conv32(x, Wr, Wroot, b):
        return aggregate(x) @ Wr.T + b + x @ Wroot.T

    pre = cat @ d0_Wr.T
    z = aggregate(pre) + cat @ d0_Wroot.T + d0_b
    x = _inorm(z, gammas[0], betas[0])
    x1 = x
    x = _inorm(conv32(x, dW_rel[0], dW_root[0], d_b[0]), gammas[1], betas[1])
    x = jax.nn.relu(x)
    x = _inorm(conv32(x, dW_rel[1], dW_root[1], d_b[1]), gammas[2], betas[2])
    x = jax.nn.relu((x + x1) / 2.0)

    for bi in (1, 2):
        i0 = 3 * bi - 1
        x = _inorm(conv32(x, dW_rel[i0], dW_root[i0], d_b[i0]),
                   gammas[3 * bi], betas[3 * bi])
        x1 = x
        x = _inorm(conv32(x, dW_rel[i0 + 1], dW_root[i0 + 1], d_b[i0 + 1]),
                   gammas[3 * bi + 1], betas[3 * bi + 1])
        x = jax.nn.relu(x)
        x = _inorm(conv32(x, dW_rel[i0 + 2], dW_root[i0 + 2], d_b[i0 + 2]),
                   gammas[3 * bi + 2], betas[3 * bi + 2])
        x = jax.nn.relu((x + x1) / 2.0)

    res = aggregate(x) @ out_Wr.T + out_b + x @ out_Wroot.T + prev_results
    return (x, res)


# speedup vs baseline: 1.0656x; 1.0459x over previous
import jax
import jax.numpy as jnp
from jax import lax
from jax.experimental import pallas as pl
from jax.experimental.pallas import tpu as pltpu
from jax.experimental.pallas import tpu_sc as plsc

N = 50000
F = 32
NPAD = 51200
E = 800000
GROUP = 408
G = 64
EPW = GROUP * G
EPAD = EPW * 32
RPT = NPAD // 16


def _agg_body(y_hbm, idx_hbm, zero_hbm, agg_hbm,
              agg_s, idx_a, idx_b, rows_a, rows_b, gsem, ssem):
    c = lax.axis_index("c")
    s = lax.axis_index("s")
    wid = c * 16 + s

    pltpu.sync_copy(zero_hbm.at[pl.ds(s * RPT, RPT)],
                    agg_s.at[pl.ds(s * RPT, RPT)])
    plsc.subcore_barrier()

    gbase = wid * G

    def pair(kk, carry):
        g = gbase + 2 * kk
        pltpu.sync_copy(idx_hbm.at[g], idx_a)
        ga = pltpu.async_copy(y_hbm.at[idx_a.at[0]], rows_a, gsem)
        pltpu.sync_copy(idx_hbm.at[g + 1], idx_b)
        gb = pltpu.async_copy(y_hbm.at[idx_b.at[0]], rows_b, gsem)
        ga.wait()
        sa = pltpu.async_copy(rows_a, agg_s.at[idx_a.at[1]], ssem, add=True)
        gb.wait()
        sb = pltpu.async_copy(rows_b, agg_s.at[idx_b.at[1]], ssem, add=True)
        sa.wait()
        sb.wait()
        return carry

    lax.fori_loop(0, G // 2, pair, 0)
    plsc.subcore_barrier()
    pltpu.sync_copy(agg_s.at[pl.ds(s * RPT, RPT)],
                    agg_hbm.at[c, pl.ds(s * RPT, RPT)])


_agg_call = pl.kernel(
    _agg_body,
    out_type=jax.ShapeDtypeStruct((2, NPAD, F), jnp.float32),
    mesh=plsc.VectorSubcoreMesh(core_axis_name="c", subcore_axis_name="s"),
    scratch_types=[
        pltpu.VMEM_SHARED((NPAD, F), jnp.float32),
        pltpu.VMEM((2, GROUP), jnp.int32),
        pltpu.VMEM((2, GROUP), jnp.int32),
        pltpu.VMEM((GROUP, F), jnp.float32),
        pltpu.VMEM((GROUP, F), jnp.float32),
        pltpu.SemaphoreType.DMA,
        pltpu.SemaphoreType.DMA,
    ],
    compiler_params=pltpu.CompilerParams(use_tc_tiling_on_sc=False),
)


def _inorm(x, g, bt):
    m = jnp.mean(x)
    v = jnp.mean((x - m) ** 2)
    return (x - m) / jnp.sqrt(v + 0.001) * g + bt


def kernel(graph_features, encoder_projection, prev_results, edge_index,
           proc_Wr, proc_Wroot, proc_b, d0_Wr, d0_Wroot, d0_b,
           dW_rel, dW_root, d_b, gammas, betas, out_Wr, out_Wroot, out_b):
    src = edge_index[0]
    dst = edge_index[1]
    pad = EPAD - E
    ar = jnp.arange(pad, dtype=jnp.int32)
    srcp = jnp.concatenate([src, (ar * 131) % N]).reshape(-1, 1, GROUP)
    dstp = jnp.concatenate([dst, N + ar % (NPAD - N)]).reshape(-1, 1, GROUP)
    idx_all = jnp.concatenate([srcp, dstp], axis=1)
    zero = jnp.zeros((NPAD, F), jnp.float32)

    def aggregate(y):
        agg2 = _agg_call(y, idx_all, zero)
        return agg2[0, :N] + agg2[1, :N]

    z = aggregate(graph_features) @ proc_Wr.T + proc_b \
        + graph_features @ proc_Wroot.T
    x0 = jax.nn.relu(z)
    cat = jnp.concatenate([x0, encoder_projection, prev_results], axis=-1)

    def conv32(x, Wr, Wroot, b):
        return aggregate(x) @ Wr.T + b + x @ Wroot.T

    pre = cat @ d0_Wr.T
    z = aggregate(pre) + cat @ d0_Wroot.T + d0_b
    x = _inorm(z, gammas[0], betas[0])
    x1 = x
    x = _inorm(conv32(x, dW_rel[0], dW_root[0], d_b[0]), gammas[1], betas[1])
    x = jax.nn.relu(x)
    x = _inorm(conv32(x, dW_rel[1], dW_root[1], d_b[1]), gammas[2], betas[2])
    x = jax.nn.relu((x + x1) / 2.0)

    for bi in (1, 2):
        i0 = 3 * bi - 1
        x = _inorm(conv32(x, dW_rel[i0], dW_root[i0], d_b[i0]),
                   gammas[3 * bi], betas[3 * bi])
        x1 = x
        x = _inorm(conv32(x, dW_rel[i0 + 1], dW_root[i0 + 1], d_b[i0 + 1]),
                   gammas[3 * bi + 1], betas[3 * bi + 1])
        x = jax.nn.relu(x)
        x = _inorm(conv32(x, dW_rel[i0 + 2], dW_root[i0 + 2], d_b[i0 + 2]),
                   gammas[3 * bi + 2], betas[3 * bi + 2])
        x = jax.nn.relu((x + x1) / 2.0)

    res = aggregate(x) @ out_Wr.T + out_b + x @ out_Wroot.T + prev_results
    return (x, res)


# speedup vs baseline: 1.0669x; 1.0012x over previous
import jax
import jax.numpy as jnp
from jax import lax
from jax.experimental import pallas as pl
from jax.experimental.pallas import tpu as pltpu
from jax.experimental.pallas import tpu_sc as plsc

N = 50000
F = 32
NPAD = 51200
E = 800000
GROUP = 408
G = 64
EPW = GROUP * G
EPAD = EPW * 32
RPT = NPAD // 16


def _agg_body(y_hbm, idx_hbm, zero_hbm, agg_hbm,
              agg_s, idx_a, idx_b, rows_a, rows_b, gsem, ssem):
    c = lax.axis_index("c")
    s = lax.axis_index("s")
    wid = c * 16 + s

    pltpu.sync_copy(zero_hbm.at[pl.ds(s * RPT, RPT)],
                    agg_s.at[pl.ds(s * RPT, RPT)])
    plsc.subcore_barrier()

    gbase = wid * G

    def pair(kk, carry):
        g = gbase + 2 * kk
        pltpu.sync_copy(idx_hbm.at[g], idx_a)
        ga = pltpu.async_copy(y_hbm.at[idx_a.at[0]], rows_a, gsem)
        pltpu.sync_copy(idx_hbm.at[g + 1], idx_b)
        gb = pltpu.async_copy(y_hbm.at[idx_b.at[0]], rows_b, gsem)
        ga.wait()
        sa = pltpu.async_copy(rows_a, agg_s.at[idx_a.at[1]], ssem, add=True)
        gb.wait()
        sb = pltpu.async_copy(rows_b, agg_s.at[idx_b.at[1]], ssem, add=True)
        sa.wait()
        sb.wait()
        return carry

    lax.fori_loop(0, G // 2, pair, 0)
    plsc.subcore_barrier()
    pltpu.sync_copy(agg_s.at[pl.ds(s * RPT, RPT)],
                    agg_hbm.at[c, pl.ds(s * RPT, RPT)])


_agg_call = pl.kernel(
    _agg_body,
    out_type=jax.ShapeDtypeStruct((2, NPAD, F), jnp.float32),
    mesh=plsc.VectorSubcoreMesh(core_axis_name="c", subcore_axis_name="s"),
    scratch_types=[
        pltpu.VMEM_SHARED((NPAD, F), jnp.float32),
        pltpu.VMEM((2, GROUP), jnp.int32),
        pltpu.VMEM((2, GROUP), jnp.int32),
        pltpu.VMEM((GROUP, F), jnp.float32),
        pltpu.VMEM((GROUP, F), jnp.float32),
        pltpu.SemaphoreType.DMA,
        pltpu.SemaphoreType.DMA,
    ],
    compiler_params=pltpu.CompilerParams(
        use_tc_tiling_on_sc=False,
        skip_device_barrier=True,
        disable_bounds_checks=True,
        disable_semaphore_checks=True,
    ),
)


def _inorm(x, g, bt):
    m = jnp.mean(x)
    v = jnp.mean((x - m) ** 2)
    return (x - m) / jnp.sqrt(v + 0.001) * g + bt


def kernel(graph_features, encoder_projection, prev_results, edge_index,
           proc_Wr, proc_Wroot, proc_b, d0_Wr, d0_Wroot, d0_b,
           dW_rel, dW_root, d_b, gammas, betas, out_Wr, out_Wroot, out_b):
    src = edge_index[0]
    dst = edge_index[1]
    pad = EPAD - E
    ar = jnp.arange(pad, dtype=jnp.int32)
    srcp = jnp.concatenate([src, (ar * 131) % N]).reshape(-1, 1, GROUP)
    dstp = jnp.concatenate([dst, N + ar % (NPAD - N)]).reshape(-1, 1, GROUP)
    idx_all = jnp.concatenate([srcp, dstp], axis=1)
    zero = jnp.zeros((NPAD, F), jnp.float32)

    def aggregate(y):
        agg2 = _agg_call(y, idx_all, zero)
        return agg2[0, :N] + agg2[1, :N]

    z = aggregate(graph_features) @ proc_Wr.T + proc_b \
        + graph_features @ proc_Wroot.T
    x0 = jax.nn.relu(z)
    cat = jnp.concatenate([x0, encoder_projection, prev_results], axis=-1)

    def conv32(x, Wr, Wroot, b):
        return aggregate(x) @ Wr.T + b + x @ Wroot.T

    pre = cat @ d0_Wr.T
    z = aggregate(pre) + cat @ d0_Wroot.T + d0_b
    x = _inorm(z, gammas[0], betas[0])
    x1 = x
    x = _inorm(conv32(x, dW_rel[0], dW_root[0], d_b[0]), gammas[1], betas[1])
    x = jax.nn.relu(x)
    x = _inorm(conv32(x, dW_rel[1], dW_root[1], d_b[1]), gammas[2], betas[2])
    x = jax.nn.relu((x + x1) / 2.0)

    for bi in (1, 2):
        i0 = 3 * bi - 1
        x = _inorm(conv32(x, dW_rel[i0], dW_root[i0], d_b[i0]),
                   gammas[3 * bi], betas[3 * bi])
        x1 = x
        x = _inorm(conv32(x, dW_rel[i0 + 1], dW_root[i0 + 1], d_b[i0 + 1]),
                   gammas[3 * bi + 1], betas[3 * bi + 1])
        x = jax.nn.relu(x)
        x = _inorm(conv32(x, dW_rel[i0 + 2], dW_root[i0 + 2], d_b[i0 + 2]),
                   gammas[3 * bi + 2], betas[3 * bi + 2])
        x = jax.nn.relu((x + x1) / 2.0)

    res = aggregate(x) @ out_Wr.T + out_b + x @ out_Wroot.T + prev_results
    return (x, res)


# speedup vs baseline: 1.0733x; 1.0061x over previous
import jax
import jax.numpy as jnp
from jax import lax
from jax.experimental import pallas as pl
from jax.experimental.pallas import tpu as pltpu
from jax.experimental.pallas import tpu_sc as plsc

N = 50000
F = 32
NPAD = 51200
E = 800000
GROUP = 384
G = 68
EPW = GROUP * G
EPAD = EPW * 32
RPT = NPAD // 16


def _agg_body(y_hbm, idx_hbm, zero_hbm, agg_hbm,
              agg_s, idxp, rows_a, rows_b, gsem, ssem):
    c = lax.axis_index("c")
    s = lax.axis_index("s")
    wid = c * 16 + s

    pltpu.sync_copy(zero_hbm.at[pl.ds(s * RPT, RPT)],
                    agg_s.at[pl.ds(s * RPT, RPT)])
    plsc.subcore_barrier()

    gbase = wid * (G // 2)

    def pair(kk, carry):
        pltpu.sync_copy(idx_hbm.at[gbase + kk], idxp)
        ga = pltpu.async_copy(y_hbm.at[idxp.at[0, 0]], rows_a, gsem)
        gb = pltpu.async_copy(y_hbm.at[idxp.at[1, 0]], rows_b, gsem)
        ga.wait()
        sa = pltpu.async_copy(rows_a, agg_s.at[idxp.at[0, 1]], ssem, add=True)
        gb.wait()
        sb = pltpu.async_copy(rows_b, agg_s.at[idxp.at[1, 1]], ssem, add=True)
        sa.wait()
        sb.wait()
        return carry

    lax.fori_loop(0, G // 2, pair, 0)
    plsc.subcore_barrier()
    pltpu.sync_copy(agg_s.at[pl.ds(s * RPT, RPT)],
                    agg_hbm.at[c, pl.ds(s * RPT, RPT)])


_agg_call = pl.kernel(
    _agg_body,
    out_type=jax.ShapeDtypeStruct((2, NPAD, F), jnp.float32),
    mesh=plsc.VectorSubcoreMesh(core_axis_name="c", subcore_axis_name="s"),
    scratch_types=[
        pltpu.VMEM_SHARED((NPAD, F), jnp.float32),
        pltpu.VMEM((2, 2, GROUP), jnp.int32),
        pltpu.VMEM((GROUP, F), jnp.float32),
        pltpu.VMEM((GROUP, F), jnp.float32),
        pltpu.SemaphoreType.DMA,
        pltpu.SemaphoreType.DMA,
    ],
    compiler_params=pltpu.CompilerParams(use_tc_tiling_on_sc=False),
)


def _inorm(x, g, bt):
    m = jnp.mean(x)
    v = jnp.mean((x - m) ** 2)
    return (x - m) / jnp.sqrt(v + 0.001) * g + bt


def kernel(graph_features, encoder_projection, prev_results, edge_index,
           proc_Wr, proc_Wroot, proc_b, d0_Wr, d0_Wroot, d0_b,
           dW_rel, dW_root, d_b, gammas, betas, out_Wr, out_Wroot, out_b):
    src = edge_index[0]
    dst = edge_index[1]
    pad = EPAD - E
    ar = jnp.arange(pad, dtype=jnp.int32)
    srcp = jnp.concatenate([src, (ar * 131) % N]).reshape(-1, 1, GROUP)
    dstp = jnp.concatenate([dst, N + ar % (NPAD - N)]).reshape(-1, 1, GROUP)
    idx_all = jnp.concatenate([srcp, dstp], axis=1).reshape(-1, 2, 2, GROUP)
    zero = jnp.zeros((NPAD, F), jnp.float32)

    def aggregate(y):
        agg2 = _agg_call(y, idx_all, zero)
        return agg2[0, :N] + agg2[1, :N]

    z = aggregate(graph_features) @ proc_Wr.T + proc_b \
        + graph_features @ proc_Wroot.T
    x0 = jax.nn.relu(z)
    cat = jnp.concatenate([x0, encoder_projection, prev_results], axis=-1)

    def conv32(x, Wr, Wroot, b):
        return aggregate(x) @ Wr.T + b + x @ Wroot.T

    pre = cat @ d0_Wr.T
    z = aggregate(pre) + cat @ d0_Wroot.T + d0_b
    x = _inorm(z, gammas[0], betas[0])
    x1 = x
    x = _inorm(conv32(x, dW_rel[0], dW_root[0], d_b[0]), gammas[1], betas[1])
    x = jax.nn.relu(x)
    x = _inorm(conv32(x, dW_rel[1], dW_root[1], d_b[1]), gammas[2], betas[2])
    x = jax.nn.relu((x + x1) / 2.0)

    for bi in (1, 2):
        i0 = 3 * bi - 1
        x = _inorm(conv32(x, dW_rel[i0], dW_root[i0], d_b[i0]),
                   gammas[3 * bi], betas[3 * bi])
        x1 = x
        x = _inorm(conv32(x, dW_rel[i0 + 1], dW_root[i0 + 1], d_b[i0 + 1]),
                   gammas[3 * bi + 1], betas[3 * bi + 1])
        x = jax.nn.relu(x)
        x = _inorm(conv32(x, dW_rel[i0 + 2], dW_root[i0 + 2], d_b[i0 + 2]),
                   gammas[3 * bi + 2], betas[3 * bi + 2])
        x = jax.nn.relu((x + x1) / 2.0)

    res = aggregate(x) @ out_Wr.T + out_b + x @ out_Wroot.T + prev_results
    return (x, res)
